# baseline (speedup 1.0000x reference)
import functools

import jax
import jax.numpy as jnp
from jax import lax
from jax.experimental import pallas as pl
from jax.experimental.pallas import tpu as pltpu
from jax.experimental.pallas import tpu_sc as plsc

VOCAB = 30522
HID = 768
PROMPT = 20
B = 32
S = 512
EPS = 1e-12
NW = 32
SW = S // NW
NBUF = 4


def _sc_gather_body(ids_hbm, word_hbm, out_hbm, idx_v, b0, b1, b2, b3,
                    g0, g1, g2, g3, w0, w1, w2, w3):
    buf = (b0, b1, b2, b3)
    gsem = (g0, g1, g2, g3)
    wsem = (w0, w1, w2, w3)

    cid = lax.axis_index("c")
    sid = lax.axis_index("s")
    wid = sid * 2 + cid
    s0 = wid * SW

    pltpu.sync_copy(ids_hbm.at[pl.ds(wid * (B * SW), B * SW)], idx_v)

    def _gather(b, k):
        return pltpu.make_async_copy(
            word_hbm.at[idx_v.at[pl.ds(b * SW, SW)]], buf[k], gsem[k])

    def _write(b, k):
        return pltpu.make_async_copy(
            buf[k], out_hbm.at[b, pl.ds(s0, SW)], wsem[k])

    _gather(0, 0).start()
    _gather(1, 1).start()

    def _quad(g, c):
        for k in range(NBUF):
            b = g * NBUF + k
            _gather(b, k).wait()
            _write(b, k).start()

            kk = (k + 2) % NBUF

            @pl.when(b + 2 < B)
            def _():
                @pl.when(b >= 2)
                def _():
                    _write(b - 2, kk).wait()
                _gather(b + 2, kk).start()
        return c
    lax.fori_loop(0, B // NBUF, _quad, 0)

    for b in range(B - NBUF, B):
        _write(b, b % NBUF).wait()


TROW = 128


def _tc_ln_body(inter_ref, pos_ref, pshift_ref, gamma_ref, beta_ref,
                out_ref):
    i = pl.program_id(1)
    x = inter_ref[0]
    grow = i * TROW + lax.broadcasted_iota(jnp.int32, (TROW, 1), 0)
    pmask = jnp.logical_and(grow >= 1, grow < 1 + PROMPT)
    x = jnp.where(pmask, pshift_ref[...], x)
    x = x + pos_ref[...]
    mean = jnp.mean(x, axis=-1, keepdims=True)
    xc = x - mean
    var = jnp.mean(xc * xc, axis=-1, keepdims=True)
    y = xc * lax.rsqrt(var + EPS)
    out_ref[0] = y * gamma_ref[...][None, :] + beta_ref[...][None, :]


def kernel(input_ids, word_emb, pos_emb, type_emb, prompt_emb, gamma, beta):
    ids_flat = (input_ids.reshape(B, NW, SW).transpose(1, 0, 2)
                .reshape(NW * B * SW))

    mesh = plsc.VectorSubcoreMesh(core_axis_name="c", subcore_axis_name="s")
    gathered = pl.kernel(
        _sc_gather_body,
        out_type=jax.ShapeDtypeStruct((B, S, HID), jnp.float32),
        mesh=mesh,
        scratch_types=(
            [pltpu.VMEM((B * SW,), jnp.int32)]
            + [pltpu.VMEM((SW, HID), jnp.float32)] * NBUF
            + [pltpu.SemaphoreType.DMA] * (2 * NBUF)
        ),
    )(ids_flat, word_emb)

    pos2 = pos_emb + type_emb[0][None, :]
    pshift = jnp.zeros((TROW, HID), jnp.float32).at[1:1 + PROMPT].set(
        prompt_emb)

    return pl.pallas_call(
        _tc_ln_body,
        out_shape=jax.ShapeDtypeStruct((B, S, HID), jnp.float32),
        grid=(B, S // TROW),
        in_specs=[
            pl.BlockSpec((1, TROW, HID), lambda b, i: (b, i, 0)),
            pl.BlockSpec((TROW, HID), lambda b, i: (i, 0)),
            pl.BlockSpec((TROW, HID), lambda b, i: (0, 0)),
            pl.BlockSpec((HID,), lambda b, i: (0,)),
            pl.BlockSpec((HID,), lambda b, i: (0,)),
        ],
        out_specs=pl.BlockSpec((1, TROW, HID), lambda b, i: (b, i, 0)),
    )(gathered, pos2, pshift, gamma, beta)


# speedup vs baseline: 1.6007x; 1.6007x over previous
import functools

import jax
import jax.numpy as jnp
from jax import lax
from jax.experimental import pallas as pl
from jax.experimental.pallas import tpu as pltpu
from jax.experimental.pallas import tpu_sc as plsc

VOCAB = 30522
HID = 768
PROMPT = 20
B = 32
S = 512
EPS = 1e-12
NW = 32
SW = S // NW
NBUF = 4


def _sc_gather_body(ids_hbm, word_hbm, out_hbm, idx_v, b0, b1, b2, b3,
                    g0, g1, g2, g3, w0, w1, w2, w3):
    buf = (b0, b1, b2, b3)
    gsem = (g0, g1, g2, g3)
    wsem = (w0, w1, w2, w3)

    cid = lax.axis_index("c")
    sid = lax.axis_index("s")
    wid = sid * 2 + cid
    s0 = wid * SW

    pltpu.sync_copy(ids_hbm.at[pl.ds(wid * (B * SW), B * SW)], idx_v)

    def _gather(b, k):
        return pltpu.make_async_copy(
            word_hbm.at[idx_v.at[pl.ds(b * SW, SW)]], buf[k], gsem[k])

    def _write(b, k):
        return pltpu.make_async_copy(
            buf[k], out_hbm.at[b, pl.ds(s0, SW)], wsem[k])

    _gather(0, 0).start()
    _gather(1, 1).start()

    def _quad(g, c):
        for k in range(NBUF):
            b = g * NBUF + k
            _gather(b, k).wait()
            _write(b, k).start()

            kk = (k + 2) % NBUF

            @pl.when(b + 2 < B)
            def _():
                @pl.when(b >= 2)
                def _():
                    _write(b - 2, kk).wait()
                _gather(b + 2, kk).start()
        return c
    lax.fori_loop(0, B // NBUF, _quad, 0)

    for b in range(B - NBUF, B):
        _write(b, b % NBUF).wait()


PBLK = 32


def _tc_ln_body(inter_ref, pos_ref, pshift_ref, gamma_ref, beta_ref,
                out_ref):
    x = inter_ref[0]
    row = lax.broadcasted_iota(jnp.int32, (PBLK, 1), 0)
    pmask = jnp.logical_and(row >= 1, row < 1 + PROMPT)
    head = jnp.where(pmask, pshift_ref[...], x[0:PBLK])
    x = jnp.concatenate([head, x[PBLK:]], axis=0)
    x = x + pos_ref[...]
    mean = jnp.mean(x, axis=-1, keepdims=True)
    xc = x - mean
    var = jnp.mean(xc * xc, axis=-1, keepdims=True)
    y = xc * lax.rsqrt(var + EPS)
    out_ref[0] = y * gamma_ref[...][None, :] + beta_ref[...][None, :]


def kernel(input_ids, word_emb, pos_emb, type_emb, prompt_emb, gamma, beta):
    ids_flat = (input_ids.reshape(B, NW, SW).transpose(1, 0, 2)
                .reshape(NW * B * SW))

    mesh = plsc.VectorSubcoreMesh(core_axis_name="c", subcore_axis_name="s")
    gathered = pl.kernel(
        _sc_gather_body,
        out_type=jax.ShapeDtypeStruct((B, S, HID), jnp.float32),
        mesh=mesh,
        scratch_types=(
            [pltpu.VMEM((B * SW,), jnp.int32)]
            + [pltpu.VMEM((SW, HID), jnp.float32)] * NBUF
            + [pltpu.SemaphoreType.DMA] * (2 * NBUF)
        ),
    )(ids_flat, word_emb)

    pos2 = pos_emb + type_emb[0][None, :]
    pshift = jnp.zeros((PBLK, HID), jnp.float32).at[1:1 + PROMPT].set(
        prompt_emb)

    return pl.pallas_call(
        _tc_ln_body,
        out_shape=jax.ShapeDtypeStruct((B, S, HID), jnp.float32),
        grid=(B,),
        in_specs=[
            pl.BlockSpec((1, S, HID), lambda b: (b, 0, 0)),
            pl.BlockSpec((S, HID), lambda b: (0, 0)),
            pl.BlockSpec((PBLK, HID), lambda b: (0, 0)),
            pl.BlockSpec((HID,), lambda b: (0,)),
            pl.BlockSpec((HID,), lambda b: (0,)),
        ],
        out_specs=pl.BlockSpec((1, S, HID), lambda b: (b, 0, 0)),
    )(gathered, pos2, pshift, gamma, beta)
